# baseline (device time: 70363 ns/iter reference)
import jax
import jax.numpy as jnp
from jax import lax
from jax.experimental import pallas as pl
from jax.experimental.pallas import tpu as pltpu

N_DEV = 4
CAP = 128
DCAP = 192


def _prefix_incl(a, rows, cols):
    cs = a
    sh = 1
    while sh < rows:
        cs = cs + jnp.concatenate(
            [jnp.zeros((sh, cols), a.dtype), cs[:-sh, :]], axis=0)
        sh *= 2
    return cs


def kernel(x, router_W, route_idx, expert_W, shared_W):
    n_tok, d = x.shape
    e_local, _, h = expert_W.shape
    blk = n_tok // N_DEV
    slots = e_local * CAP

    def body(x_ref, rw_ref, ridx_ref, ew_hbm, sw_ref, out_ref,
             wbuf, p_ref, ye_ref, dh_ref, send_ref, recv_ref,
             wsems, send_sems, recv_sems):
        me = lax.axis_index("i")

        barrier = pltpu.get_barrier_semaphore()
        for off in range(1, N_DEV):
            peer = lax.rem(me + off, N_DEV)
            pl.semaphore_signal(barrier, inc=1, device_id=(peer,),
                                device_id_type=pl.DeviceIdType.MESH)
        pl.semaphore_wait(barrier, N_DEV - 1)

        pltpu.make_async_copy(ew_hbm.at[0], wbuf.at[0], wsems.at[0]).start()

        sc = jnp.dot(x_ref[:, :], rw_ref[:, :],
                     preferred_element_type=jnp.float32)
        sc = sc - jnp.max(sc, axis=-1, keepdims=True)
        p = jnp.exp(sc)
        probs = p / jnp.sum(p, axis=-1, keepdims=True)
        ridx = ridx_ref[:, :]
        cols = lax.broadcasted_iota(jnp.int32, probs.shape, 1)
        chosen = jnp.sum(jnp.where(cols == ridx, probs, 0.0),
                         axis=1, keepdims=True)

        el = lax.broadcasted_iota(jnp.int32, (n_tok, e_local), 1) \
            + me * e_local
        m = (ridx == el).astype(jnp.float32)
        cs = _prefix_incl(m, n_tok, e_local)
        ranks = cs - m
        rk = jnp.sum(ranks * m, axis=1, keepdims=True).astype(jnp.int32)
        target_e = ridx - me * e_local

        colid = lax.broadcasted_iota(jnp.int32, (n_tok, slots), 1)
        p_all = jnp.where(
            (jnp.right_shift(colid, 7) == target_e)
            & (jnp.bitwise_and(colid, CAP - 1) == rk),
            1.0, 0.0)
        p_ref[:, :] = p_all

        xe = lax.dot_general(p_all, x_ref[:, :],
                             (((0,), (0,)), ((), ())),
                             preferred_element_type=jnp.float32)
        pe = lax.dot_general(p_all, chosen,
                             (((0,), (0,)), ((), ())),
                             preferred_element_type=jnp.float32)

        for e in range(e_local):
            pltpu.make_async_copy(
                ew_hbm.at[e], wbuf.at[e % 2], wsems.at[e % 2]).wait()
            if e + 1 < e_local:
                pltpu.make_async_copy(
                    ew_hbm.at[e + 1], wbuf.at[(e + 1) % 2],
                    wsems.at[(e + 1) % 2]).start()
            lo = e * CAP
            ye_ref[lo:lo + CAP, :] = pe[lo:lo + CAP, :] * jnp.dot(
                xe[lo:lo + CAP, :], wbuf[e % 2, :, :],
                preferred_element_type=jnp.float32)

        mloc = jnp.sum(m, axis=1, keepdims=True)
        cl = jnp.sum(cs, axis=1, keepdims=True)
        segs = []
        for jj in range(N_DEV):
            lo = jj * blk
            seg = cl[lo:lo + blk, :] - mloc[lo:lo + blk, :]
            segs.append(seg - (cl[lo:lo + 1, :] - mloc[lo:lo + 1, :]))
        dr = jnp.concatenate(segs, axis=0).astype(jnp.int32)

        dcol = lax.broadcasted_iota(jnp.int32, (n_tok, DCAP), 1)
        dh_ref[:, :] = jnp.where((dcol == dr) & (mloc > 0.0), 1.0, 0.0)

        chipof = jnp.right_shift(ridx_ref[pl.ds(me * blk, blk), :], 3)
        ccol = lax.broadcasted_iota(jnp.int32, (blk, N_DEV), 1)
        m4 = (chipof == ccol).astype(jnp.float32)
        cs4 = _prefix_incl(m4, blk, N_DEV) - m4

        rdmas = []
        for s in range(1, N_DEV):
            j = lax.rem(me + s, N_DEV)
            row = pl.ds(j * blk, blk)
            g = lax.dot_general(dh_ref[row, :], p_ref[row, :],
                                (((0,), (0,)), ((), ())),
                                preferred_element_type=jnp.float32)
            send_ref[s - 1, :, :] = jnp.dot(
                g, ye_ref[:, :], preferred_element_type=jnp.float32)
            rdma = pltpu.make_async_remote_copy(
                src_ref=send_ref.at[s - 1],
                dst_ref=recv_ref.at[N_DEV - 1 - s],
                send_sem=send_sems.at[s - 1],
                recv_sem=recv_sems.at[N_DEV - 1 - s],
                device_id=(j,),
                device_id_type=pl.DeviceIdType.MESH,
            )
            rdma.start()
            rdmas.append(rdma)

        own_row = pl.ds(me * blk, blk)
        out_ref[:, :] = jnp.dot(
            p_ref[own_row, :], ye_ref[:, :],
            preferred_element_type=jnp.float32,
        ) + jnp.dot(x_ref[own_row, :], sw_ref[:, :],
                    preferred_element_type=jnp.float32)

        dcol_b = lax.broadcasted_iota(jnp.int32, (blk, DCAP), 1)
        for sig in range(N_DEV - 2, -1, -1):
            k = lax.rem(me + sig + 1, N_DEV)
            ck = (ccol == k).astype(jnp.float32)
            mk = jnp.sum(m4 * ck, axis=1, keepdims=True)
            drk = jnp.sum(cs4 * ck, axis=1, keepdims=True).astype(jnp.int32)
            rhot = jnp.where((dcol_b == drk) & (mk > 0.0), 1.0, 0.0)
            rdmas[2 - sig].wait_recv()
            out_ref[:, :] = out_ref[:, :] + jnp.dot(
                rhot, recv_ref[sig, :, :],
                preferred_element_type=jnp.float32)
        for rdma in rdmas:
            rdma.wait_send()

    return pl.pallas_call(
        body,
        out_shape=jax.ShapeDtypeStruct((blk, h), jnp.float32),
        in_specs=[
            pl.BlockSpec(memory_space=pltpu.VMEM),
            pl.BlockSpec(memory_space=pltpu.VMEM),
            pl.BlockSpec(memory_space=pltpu.VMEM),
            pl.BlockSpec(memory_space=pltpu.MemorySpace.HBM),
            pl.BlockSpec(memory_space=pltpu.VMEM),
        ],
        out_specs=pl.BlockSpec(memory_space=pltpu.VMEM),
        scratch_shapes=[
            pltpu.VMEM((2, d, h), jnp.float32),
            pltpu.VMEM((n_tok, slots), jnp.float32),
            pltpu.VMEM((slots, h), jnp.float32),
            pltpu.VMEM((n_tok, DCAP), jnp.float32),
            pltpu.VMEM((N_DEV - 1, DCAP, h), jnp.float32),
            pltpu.VMEM((N_DEV - 1, DCAP, h), jnp.float32),
            pltpu.SemaphoreType.DMA((2,)),
            pltpu.SemaphoreType.DMA((N_DEV - 1,)),
            pltpu.SemaphoreType.DMA((N_DEV - 1,)),
        ],
        compiler_params=pltpu.CompilerParams(
            collective_id=0,
            vmem_limit_bytes=100 * 1024 * 1024,
            fuse_transposed_lhs_in_matmul=True,
        ),
    )(x, router_W, route_idx, expert_W, shared_W)
